# baseline (speedup 1.0000x reference)
import functools

import jax
import jax.numpy as jnp
from jax import lax
from jax.experimental import pallas as pl
from jax.experimental.pallas import tpu as pltpu
from jax.experimental.pallas import tpu_sc as plsc

NC = 2
NS = 16
NW = NC * NS
E = 1536
U = 16


def _build(n_rows, k_dim, steps, w_tail):
    mesh = plsc.VectorSubcoreMesh(core_axis_name="c", subcore_axis_name="s")
    per_w = steps * E

    @functools.partial(
        pl.kernel,
        out_type=jax.ShapeDtypeStruct((NW, k_dim), jnp.float32),
        mesh=mesh,
        compiler_params=pltpu.CompilerParams(use_tc_tiling_on_sc=False),
        scratch_types=[
            pltpu.VMEM((E,), jnp.int32),
            pltpu.VMEM((E,), jnp.int32),
            pltpu.VMEM((E,), jnp.int32),
            pltpu.VMEM((E,), jnp.int32),
            pltpu.VMEM((E,), jnp.float32),
            pltpu.VMEM((E,), jnp.float32),
            pltpu.VMEM((E, k_dim), jnp.float32),
            pltpu.VMEM((E, k_dim), jnp.float32),
            pltpu.VMEM((E, k_dim), jnp.float32),
            pltpu.VMEM((E, k_dim), jnp.float32),
            pltpu.VMEM((k_dim,), jnp.float32),
            pltpu.VMEM_SHARED((n_rows, k_dim), jnp.float32),
            pltpu.SemaphoreType.DMA,
            pltpu.SemaphoreType.DMA,
            pltpu.SemaphoreType.DMA,
            pltpu.SemaphoreType.DMA,
        ],
    )
    def k(prob_h, rows_h, cols_h, vals_h, tr_h, tc_h, tv_h, out_h,
          r0, r1, c0, c1, v0, v1, pr0, pr1, pc0, pc1, accv, tab_s,
          si0, si1, sg0, sg1):
        cid = lax.axis_index("c")
        sid = lax.axis_index("s")
        wid = sid * NC + cid
        base = pl.multiple_of(wid * per_w, 8)

        def stage_table():
            @pl.when(sid == 0)
            def _():
                pltpu.sync_copy(prob_h, tab_s)

            plsc.subcore_barrier()

        rbuf = (r0, r1)
        cbuf = (c0, c1)
        vbuf = (v0, v1)
        prb = (pr0, pr1)
        pcb = (pc0, pc1)
        sin = (si0, si1)
        sg = (sg0, sg1)

        def issue_in(s, b):
            @pl.when(wid < w_tail)
            def _():
                off = pl.multiple_of(base + s * E, 8)
                pltpu.async_copy(rows_h.at[pl.ds(off, E)], rbuf[b], sin[b])
                pltpu.async_copy(cols_h.at[pl.ds(off, E)], cbuf[b], sin[b])
                pltpu.async_copy(vals_h.at[pl.ds(off, E)], vbuf[b], sin[b])

            @pl.when(wid >= w_tail)
            def _():
                off = pl.multiple_of((wid - w_tail) * per_w + s * E, 8)
                pltpu.async_copy(tr_h.at[pl.ds(off, E)], rbuf[b], sin[b])
                pltpu.async_copy(tc_h.at[pl.ds(off, E)], cbuf[b], sin[b])
                pltpu.async_copy(tv_h.at[pl.ds(off, E)], vbuf[b], sin[b])

        def wait_in(b):
            pltpu.make_async_copy(rows_h.at[pl.ds(0, E)], rbuf[b], sin[b]).wait()
            pltpu.make_async_copy(cols_h.at[pl.ds(0, E)], cbuf[b], sin[b]).wait()
            pltpu.make_async_copy(vals_h.at[pl.ds(0, E)], vbuf[b], sin[b]).wait()

        def issue_gather(b):
            pltpu.async_copy(tab_s.at[rbuf[b]], prb[b], sg[b])
            pltpu.async_copy(tab_s.at[cbuf[b]], pcb[b], sg[b])

        def wait_gather(b):
            pltpu.make_async_copy(tab_s.at[rbuf[b]], prb[b], sg[b]).wait()
            pltpu.make_async_copy(tab_s.at[cbuf[b]], pcb[b], sg[b]).wait()

        def compute(b, accs):
            pr, pc, vv = prb[b], pcb[b], vbuf[b]

            @plsc.parallel_loop(0, E, step=U, carry=accs)
            def done(i, a):
                vvec = vv[pl.ds(i, U)]
                out = []
                for u in range(U):
                    t = pr[i + u, :] * pc[i + u, :]
                    out.append(a[u] + vvec[u] * t)
                return tuple(out)

            return done

        def one_step(s, b, accs):
            wait_in(1 - b)
            issue_gather(1 - b)
            wait_gather(b)
            accs = compute(b, accs)
            issue_in(s + 2, b)
            return accs

        accs0 = tuple(jnp.zeros((k_dim,), jnp.float32) for _ in range(U))

        issue_in(0, 0)
        issue_in(1, 1)
        stage_table()
        wait_in(0)
        issue_gather(0)
        accs0 = one_step(jnp.int32(0), 0, accs0)

        def body2(i2, accs):
            s = 2 * i2 + 1
            accs = one_step(s, 1, accs)
            accs = one_step(s + 1, 0, accs)
            return accs

        accs0 = lax.fori_loop(0, (steps - 1) // 2, body2, accs0)

        wait_in((steps + 1) % 2)
        wait_gather(steps % 2)

        total = accs0[0]
        for u in range(1, U):
            total = total + accs0[u]
        accv[...] = total
        pltpu.sync_copy(accv, out_h.at[wid])

    return k


def kernel(prob, mat_vals, mat_rows, mat_cols, num_edges):
    nnz = mat_vals.shape[0]
    n_rows, k_dim = prob.shape
    per_w = -(-nnz // NW)
    steps = -(-per_w // E)
    if steps % 2 == 0:
        steps += 1
    per_w = steps * E
    w_tail = NW
    while w_tail > 0 and (w_tail - 1 + 1) * per_w + 2 * E > nnz:
        w_tail -= 1
    start = w_tail * per_w
    avail = nnz - start
    tail_len = (NW - w_tail) * per_w + 2 * E
    rows = mat_rows.astype(jnp.int32)
    cols = mat_cols.astype(jnp.int32)

    def tail(x):
        t = lax.slice(x, (start,), (nnz,))
        return jnp.pad(t, (0, tail_len - avail))

    tr = tail(rows)
    tcl = tail(cols)
    tv = tail(mat_vals)

    partials = _build(n_rows, k_dim, steps, w_tail)(
        prob, rows, cols, mat_vals, tr, tcl, tv)
    result = jnp.sum(partials)
    return jnp.reshape(result, (1,)) / num_edges


# speedup vs baseline: 1.3979x; 1.3979x over previous
import functools

import jax
import jax.numpy as jnp
from jax import lax
from jax.experimental import pallas as pl
from jax.experimental.pallas import tpu as pltpu
from jax.experimental.pallas import tpu_sc as plsc

NC = 2
NS = 16
NW = NC * NS
E = 1024
U = 16


def _build(n_rows, k_dim, steps, w_tail):
    mesh = plsc.VectorSubcoreMesh(core_axis_name="c", subcore_axis_name="s")
    per_w = steps * E

    @functools.partial(
        pl.kernel,
        out_type=jax.ShapeDtypeStruct((NW, k_dim), jnp.float32),
        mesh=mesh,
        compiler_params=pltpu.CompilerParams(use_tc_tiling_on_sc=False),
        scratch_types=[
            pltpu.VMEM((E,), jnp.int32),
            pltpu.VMEM((E,), jnp.int32),
            pltpu.VMEM((E,), jnp.int32),
            pltpu.VMEM((E,), jnp.int32),
            pltpu.VMEM((E,), jnp.float32),
            pltpu.VMEM((E,), jnp.float32),
            pltpu.VMEM((E, k_dim), jnp.float32),
            pltpu.VMEM((E, k_dim), jnp.float32),
            pltpu.VMEM((E, k_dim), jnp.float32),
            pltpu.VMEM((E, k_dim), jnp.float32),
            pltpu.VMEM((k_dim,), jnp.float32),
            pltpu.VMEM_SHARED((n_rows, k_dim), jnp.float32),
            pltpu.SemaphoreType.DMA,
            pltpu.SemaphoreType.DMA,
            pltpu.SemaphoreType.DMA,
            pltpu.SemaphoreType.DMA,
        ],
    )
    def k(prob_h, rows_h, cols_h, vals_h, tr_h, tc_h, tv_h, out_h,
          r0, r1, c0, c1, v0, v1, pr0, pr1, pc0, pc1, accv, tab_s,
          si0, si1, sg0, sg1):
        cid = lax.axis_index("c")
        sid = lax.axis_index("s")
        wid = sid * NC + cid
        base = pl.multiple_of(wid * per_w, 8)

        def stage_table():
            @pl.when(sid == 0)
            def _():
                pltpu.sync_copy(prob_h, tab_s)

            plsc.subcore_barrier()

        rbuf = (r0, r1)
        cbuf = (c0, c1)
        vbuf = (v0, v1)
        prb = (pr0, pr1)
        pcb = (pc0, pc1)
        sin = (si0, si1)
        sg = (sg0, sg1)

        def issue_in(s, b):
            @pl.when(wid < w_tail)
            def _():
                off = pl.multiple_of(base + s * E, 8)
                pltpu.async_copy(rows_h.at[pl.ds(off, E)], rbuf[b], sin[b])
                pltpu.async_copy(cols_h.at[pl.ds(off, E)], cbuf[b], sin[b])
                pltpu.async_copy(vals_h.at[pl.ds(off, E)], vbuf[b], sin[b])

            @pl.when(wid >= w_tail)
            def _():
                off = pl.multiple_of((wid - w_tail) * per_w + s * E, 8)
                pltpu.async_copy(tr_h.at[pl.ds(off, E)], rbuf[b], sin[b])
                pltpu.async_copy(tc_h.at[pl.ds(off, E)], cbuf[b], sin[b])
                pltpu.async_copy(tv_h.at[pl.ds(off, E)], vbuf[b], sin[b])

        def wait_in(b):
            pltpu.make_async_copy(rows_h.at[pl.ds(0, E)], rbuf[b], sin[b]).wait()
            pltpu.make_async_copy(cols_h.at[pl.ds(0, E)], cbuf[b], sin[b]).wait()
            pltpu.make_async_copy(vals_h.at[pl.ds(0, E)], vbuf[b], sin[b]).wait()

        def issue_gather(b):
            pltpu.async_copy(tab_s.at[rbuf[b]], prb[b], sg[b])
            pltpu.async_copy(tab_s.at[cbuf[b]], pcb[b], sg[b])

        def wait_gather(b):
            pltpu.make_async_copy(tab_s.at[rbuf[b]], prb[b], sg[b]).wait()
            pltpu.make_async_copy(tab_s.at[cbuf[b]], pcb[b], sg[b]).wait()

        def compute(b, accs):
            pr, pc, vv = prb[b], pcb[b], vbuf[b]

            @plsc.parallel_loop(0, E, step=U, carry=accs)
            def done(i, a):
                vvec = vv[pl.ds(i, U)]
                out = []
                for u in range(U):
                    t = pr[i + u, :] * pc[i + u, :]
                    out.append(a[u] + vvec[u] * t)
                return tuple(out)

            return done

        def one_step(s, b, accs):
            wait_in(1 - b)
            issue_gather(1 - b)
            wait_gather(b)
            accs = compute(b, accs)
            issue_in(s + 2, b)
            return accs

        accs0 = tuple(jnp.zeros((k_dim,), jnp.float32) for _ in range(U))

        issue_in(0, 0)
        issue_in(1, 1)
        stage_table()
        wait_in(0)
        issue_gather(0)
        accs0 = one_step(jnp.int32(0), 0, accs0)

        def body2(i2, accs):
            s = 2 * i2 + 1
            accs = one_step(s, 1, accs)
            accs = one_step(s + 1, 0, accs)
            return accs

        accs0 = lax.fori_loop(0, (steps - 1) // 2, body2, accs0)

        wait_in((steps + 1) % 2)
        wait_gather(steps % 2)

        total = accs0[0]
        for u in range(1, U):
            total = total + accs0[u]
        accv[...] = total
        pltpu.sync_copy(accv, out_h.at[wid])

    return k


def kernel(prob, mat_vals, mat_rows, mat_cols, num_edges):
    nnz = mat_vals.shape[0]
    n_rows, k_dim = prob.shape
    per_w = -(-nnz // NW)
    steps = -(-per_w // E)
    if steps % 2 == 0:
        steps += 1
    per_w = steps * E
    w_tail = NW
    while w_tail > 0 and (w_tail - 1 + 1) * per_w + 2 * E > nnz:
        w_tail -= 1
    start = w_tail * per_w
    avail = nnz - start
    tail_len = (NW - w_tail) * per_w + 2 * E
    rows = mat_rows.astype(jnp.int32)
    cols = mat_cols.astype(jnp.int32)

    def tail(x):
        t = lax.slice(x, (start,), (nnz,))
        return jnp.pad(t, (0, tail_len - avail))

    tr = tail(rows)
    tcl = tail(cols)
    tv = tail(mat_vals)

    partials = _build(n_rows, k_dim, steps, w_tail)(
        prob, rows, cols, mat_vals, tr, tcl, tv)
    result = jnp.sum(partials)
    return jnp.reshape(result, (1,)) / num_edges
